# baseline (speedup 1.0000x reference)
import functools

import jax
import jax.numpy as jnp
from jax import lax
from jax.experimental import pallas as pl
from jax.experimental.pallas import tpu as pltpu

_CompilerParams = getattr(pltpu, "CompilerParams", None)
if _CompilerParams is None:
    _CompilerParams = pltpu.TPUCompilerParams

_ANY = getattr(pl, "ANY", None)
if _ANY is None:
    _ANY = pltpu.MemorySpace.HBM

B_BLK = 64
_UNROLL = 16


def _ncf_kernel(
    item_idx_ref,
    user_idx_ref,
    wi_ref,
    wug_ref,
    wum_ref,
    bug_ref, bum_ref, big_ref, bim_ref,
    w1_ref, b1_ref, w2_ref, b2_ref, w3_ref, b3_ref, wp_ref, bp_ref,
    out_ref,
    scr_i,
    scr_ug, scr_um,
    r_scr,
    idx_smem,
    uidx_smem,
    sem_si, sem_su, sem_i, sem_ug, sem_um,
    *, n_items: int, nb: int,
):
    m_rows = B_BLK * n_items
    step = pl.program_id(0)

    pltpu.make_async_copy(item_idx_ref, idx_smem, sem_si).start(priority=1)
    pltpu.make_async_copy(user_idx_ref, uidx_smem, sem_su).start(priority=1)
    pltpu.make_async_copy(user_idx_ref, uidx_smem, sem_su).wait()

    for u in range(B_BLK):
        t = uidx_smem[0, 0, u]
        pltpu.make_async_copy(wug_ref.at[t], scr_ug.at[u, 0], sem_ug).start()
        pltpu.make_async_copy(wum_ref.at[t], scr_um.at[u, 0], sem_um).start()

    pltpu.make_async_copy(item_idx_ref, idx_smem, sem_si).wait()

    def issue_chunk(c, _):
        base = c * _UNROLL
        for i in range(_UNROLL):
            k = base + i
            t = idx_smem[0, 0, k]
            pltpu.make_async_copy(wi_ref.at[t], scr_i.at[k, 0], sem_i).start()
        return ()
    lax.fori_loop(0, m_rows // _UNROLL, issue_chunk, ())

    @pl.when(step == 0)
    def _():
        k_io = lax.broadcasted_iota(jnp.int32, (m_rows, B_BLK), 0)
        u_io = lax.broadcasted_iota(jnp.int32, (m_rows, B_BLK), 1) * n_items
        r_scr[...] = ((k_io >= u_io) & (k_io < u_io + n_items)).astype(jnp.float32)

    pltpu.make_async_copy(scr_ug, scr_ug, sem_ug).wait()
    pltpu.make_async_copy(scr_um, scr_um, sem_um).wait()
    eu_g = scr_ug.reshape(B_BLK, 64)[...] + bug_ref[...]
    eu_m = scr_um.reshape(B_BLK, 64)[...] + bum_ref[...]

    w1 = w1_ref[...]
    u1 = jnp.dot(eu_m, w1[0:64, :], preferred_element_type=jnp.float32)

    r_mat = r_scr[...]
    eu_g_rep = jnp.dot(r_mat, eu_g, preferred_element_type=jnp.float32)
    u1_rep = jnp.dot(r_mat, u1, preferred_element_type=jnp.float32)

    pltpu.make_async_copy(scr_i, scr_i, sem_i).wait()
    packed = scr_i.reshape(m_rows, 64)[...]
    ei_g = lax.bitcast_convert_type(packed << 16, jnp.float32) + big_ref[...]
    ei_m = lax.bitcast_convert_type(packed & jnp.int32(-65536), jnp.float32) + bim_ref[...]

    gmf = eu_g_rep * ei_g
    i1 = jnp.dot(ei_m, w1[64:128, :], preferred_element_type=jnp.float32)
    h1 = jnp.maximum(u1_rep + i1 + b1_ref[...], 0.0)
    h2 = jnp.maximum(
        jnp.dot(h1, w2_ref[...], preferred_element_type=jnp.float32) + b2_ref[...], 0.0)
    h3 = jnp.maximum(
        jnp.dot(h2, w3_ref[...], preferred_element_type=jnp.float32) + b3_ref[...], 0.0)

    wp = wp_ref[...]
    logit = (jnp.dot(gmf, wp[0:64, :], preferred_element_type=jnp.float32)
             + jnp.dot(h3, wp[64:96, :], preferred_element_type=jnp.float32)
             + bp_ref[...])
    out_ref[...] = jax.nn.sigmoid(logit)


def kernel(user, item, num_total, Wu_gmf, bu_gmf, Wu_mlp, bu_mlp,
           Wi_gmf, bi_gmf, Wi_mlp, bi_mlp, W1, b1, W2, b2, W3, b3, Wp, bp):
    batch, n_items = item.shape
    nb = batch // B_BLK
    m_rows = B_BLK * n_items
    embed = Wu_gmf.shape[1]

    item_idx = item.astype(jnp.int32).reshape(nb, 1, m_rows)
    user_idx = user.astype(jnp.int32).reshape(nb, 1, B_BLK)

    g16 = lax.bitcast_convert_type(Wi_gmf.astype(jnp.bfloat16), jnp.uint16)
    m16 = lax.bitcast_convert_type(Wi_mlp.astype(jnp.bfloat16), jnp.uint16)
    wi_pack = (g16.astype(jnp.uint32) | (m16.astype(jnp.uint32) << 16)).astype(jnp.int32)

    biases = [b.reshape(1, -1) for b in (bu_gmf, bu_mlp, bi_gmf, bi_mlp, b1, b2, b3)]
    bp2 = bp.reshape(1, 1)

    in_specs = [
            pl.BlockSpec((1, 1, m_rows), lambda i: (i, 0, 0)),
            pl.BlockSpec((1, 1, B_BLK), lambda i: (i, 0, 0)),
            pl.BlockSpec(memory_space=_ANY),
            pl.BlockSpec(memory_space=_ANY),
            pl.BlockSpec(memory_space=_ANY),
            pl.BlockSpec((1, embed), lambda i: (0, 0)),
            pl.BlockSpec((1, embed), lambda i: (0, 0)),
            pl.BlockSpec((1, embed), lambda i: (0, 0)),
            pl.BlockSpec((1, embed), lambda i: (0, 0)),
            pl.BlockSpec(W1.shape, lambda i: (0, 0)),
            pl.BlockSpec((1, 2 * embed), lambda i: (0, 0)),
            pl.BlockSpec(W2.shape, lambda i: (0, 0)),
            pl.BlockSpec((1, embed), lambda i: (0, 0)),
            pl.BlockSpec(W3.shape, lambda i: (0, 0)),
            pl.BlockSpec((1, embed // 2), lambda i: (0, 0)),
            pl.BlockSpec(Wp.shape, lambda i: (0, 0)),
            pl.BlockSpec((1, 1), lambda i: (0, 0)),
    ]

    pred = pl.pallas_call(
        functools.partial(_ncf_kernel, n_items=n_items, nb=nb),
        out_shape=jax.ShapeDtypeStruct((batch * n_items, 1), jnp.float32),
        grid=(nb,),
        in_specs=in_specs,
        out_specs=pl.BlockSpec((m_rows, 1), lambda i: (i, 0)),
        scratch_shapes=[
            pltpu.VMEM((m_rows, 1, embed), jnp.int32),
            pltpu.VMEM((B_BLK, 1, embed), jnp.float32),
            pltpu.VMEM((B_BLK, 1, embed), jnp.float32),
            pltpu.VMEM((m_rows, B_BLK), jnp.float32),
            pltpu.SMEM((1, 1, m_rows), jnp.int32),
            pltpu.SMEM((1, 1, B_BLK), jnp.int32),
            pltpu.SemaphoreType.DMA,
            pltpu.SemaphoreType.DMA,
            pltpu.SemaphoreType.DMA,
            pltpu.SemaphoreType.DMA,
            pltpu.SemaphoreType.DMA,
        ],
        compiler_params=_CompilerParams(
            dimension_semantics=("arbitrary",),
        ),
        name="ncf_fused",
    )(item_idx, user_idx, wi_pack, Wu_gmf, Wu_mlp, biases[0], biases[1],
      biases[2], biases[3], W1, biases[4], W2, biases[5], W3, biases[6], Wp, bp2)

    return pred.reshape(batch, n_items)


# speedup vs baseline: 1.0092x; 1.0092x over previous
import functools

import jax
import jax.numpy as jnp
from jax import lax
from jax.experimental import pallas as pl
from jax.experimental.pallas import tpu as pltpu

_CompilerParams = getattr(pltpu, "CompilerParams", None)
if _CompilerParams is None:
    _CompilerParams = pltpu.TPUCompilerParams

_ANY = getattr(pl, "ANY", None)
if _ANY is None:
    _ANY = pltpu.MemorySpace.HBM

B_BLK = 64
_UNROLL = 16


def _ncf_kernel(
    item_idx_ref,
    user_idx_ref,
    wi_ref,
    wug_ref,
    wum_ref,
    bug_ref, bum_ref, big_ref, bim_ref,
    w1_ref, b1_ref, w2_ref, b2_ref, w3_ref, b3_ref, wp_ref, bp_ref,
    out_ref,
    scr_i,
    scr_ug, scr_um,
    r_scr,
    idx_smem,
    uidx_smem,
    sem_si, sem_su, sem_i, sem_ug, sem_um,
    *, n_items: int, nb: int,
):
    m_rows = B_BLK * n_items
    step = pl.program_id(0)
    cur = lax.rem(step, 2)
    nxt = lax.rem(step + 1, 2)

    def stage_idx(b, slot):
        pltpu.make_async_copy(item_idx_ref.at[b], idx_smem.at[slot], sem_si).start(
            priority=1)
        pltpu.make_async_copy(user_idx_ref.at[b], uidx_smem.at[slot], sem_su).start(
            priority=1)

    def wait_idx(slot):
        pltpu.make_async_copy(item_idx_ref.at[0], idx_smem.at[slot], sem_si).wait()
        pltpu.make_async_copy(user_idx_ref.at[0], uidx_smem.at[slot], sem_su).wait()

    def issue_items(slot):
        def issue_chunk(c, _):
            base = c * _UNROLL
            for i in range(_UNROLL):
                k = base + i
                t = idx_smem[slot, 0, k]
                pltpu.make_async_copy(
                    wi_ref.at[t], scr_i.at[slot, k, 0], sem_i.at[slot]).start()
            return ()
        lax.fori_loop(0, m_rows // _UNROLL, issue_chunk, ())

    def issue_users(slot):
        for u in range(B_BLK):
            t = uidx_smem[slot, 0, u]
            pltpu.make_async_copy(
                wug_ref.at[t], scr_ug.at[slot, u, 0], sem_ug.at[slot]).start()
            pltpu.make_async_copy(
                wum_ref.at[t], scr_um.at[slot, u, 0], sem_um.at[slot]).start()

    def issue_gathers(slot_dyn):
        issue_users(slot_dyn)
        @pl.when(slot_dyn == 0)
        def _():
            issue_items(0)
        @pl.when(slot_dyn == 1)
        def _():
            issue_items(1)

    @pl.when(step == 0)
    def _():
        stage_idx(0, 0)
        wait_idx(0)
        issue_users(0)
        issue_items(0)
        k_io = lax.broadcasted_iota(jnp.int32, (m_rows, B_BLK), 0)
        u_io = lax.broadcasted_iota(jnp.int32, (m_rows, B_BLK), 1) * n_items
        r_scr[...] = ((k_io >= u_io) & (k_io < u_io + n_items)).astype(jnp.float32)

    @pl.when(step + 1 < nb)
    def _():
        stage_idx(step + 1, nxt)

    pltpu.make_async_copy(scr_ug.at[cur], scr_ug.at[cur], sem_ug.at[cur]).wait()
    pltpu.make_async_copy(scr_um.at[cur], scr_um.at[cur], sem_um.at[cur]).wait()
    eu_g = scr_ug.reshape(2, B_BLK, 64).at[cur][...] + bug_ref[...]
    eu_m = scr_um.reshape(2, B_BLK, 64).at[cur][...] + bum_ref[...]

    w1 = w1_ref[...]
    u1 = jnp.dot(eu_m, w1[0:64, :], preferred_element_type=jnp.float32)

    r_mat = r_scr[...]
    eu_g_rep = jnp.dot(r_mat, eu_g, preferred_element_type=jnp.float32)
    u1_rep = jnp.dot(r_mat, u1, preferred_element_type=jnp.float32)

    @pl.when(step + 1 < nb)
    def _():
        wait_idx(nxt)
        issue_gathers(nxt)

    pltpu.make_async_copy(scr_i.at[cur], scr_i.at[cur], sem_i.at[cur]).wait()
    packed = scr_i.reshape(2, m_rows, 64).at[cur][...]
    ei_g = lax.bitcast_convert_type(packed << 16, jnp.float32) + big_ref[...]
    ei_m = lax.bitcast_convert_type(packed & jnp.int32(-65536), jnp.float32) + bim_ref[...]

    gmf = eu_g_rep * ei_g
    i1 = jnp.dot(ei_m, w1[64:128, :], preferred_element_type=jnp.float32)
    h1 = jnp.maximum(u1_rep + i1 + b1_ref[...], 0.0)
    h2 = jnp.maximum(
        jnp.dot(h1, w2_ref[...], preferred_element_type=jnp.float32) + b2_ref[...], 0.0)
    h3 = jnp.maximum(
        jnp.dot(h2, w3_ref[...], preferred_element_type=jnp.float32) + b3_ref[...], 0.0)

    wp = wp_ref[...]
    logit = (jnp.dot(gmf, wp[0:64, :], preferred_element_type=jnp.float32)
             + jnp.dot(h3, wp[64:96, :], preferred_element_type=jnp.float32)
             + bp_ref[...])
    out_ref[...] = jax.nn.sigmoid(logit)


def _pack_bf16_pair(lo_f32, hi_f32):
    lo = lax.bitcast_convert_type(lo_f32, jnp.uint32)
    hi = lax.bitcast_convert_type(hi_f32, jnp.uint32)
    lo16 = (lo + jnp.uint32(0x7FFF) + ((lo >> 16) & jnp.uint32(1))) >> 16
    hi16 = (hi + jnp.uint32(0x7FFF) + ((hi >> 16) & jnp.uint32(1))) & jnp.uint32(0xFFFF0000)
    return lax.bitcast_convert_type(lo16 | hi16, jnp.int32)


def kernel(user, item, num_total, Wu_gmf, bu_gmf, Wu_mlp, bu_mlp,
           Wi_gmf, bi_gmf, Wi_mlp, bi_mlp, W1, b1, W2, b2, W3, b3, Wp, bp):
    batch, n_items = item.shape
    nb = batch // B_BLK
    m_rows = B_BLK * n_items
    embed = Wu_gmf.shape[1]

    item_idx = item.astype(jnp.int32).reshape(nb, 1, m_rows)
    user_idx = user.astype(jnp.int32).reshape(nb, 1, B_BLK)

    wi_pack = _pack_bf16_pair(Wi_gmf, Wi_mlp)

    biases = [b.reshape(1, -1) for b in (bu_gmf, bu_mlp, bi_gmf, bi_mlp, b1, b2, b3)]
    bp2 = bp.reshape(1, 1)

    in_specs = [
            pl.BlockSpec((nb, 1, m_rows), lambda i: (0, 0, 0)),
            pl.BlockSpec((nb, 1, B_BLK), lambda i: (0, 0, 0)),
            pl.BlockSpec(memory_space=_ANY),
            pl.BlockSpec(memory_space=_ANY),
            pl.BlockSpec(memory_space=_ANY),
            pl.BlockSpec((1, embed), lambda i: (0, 0)),
            pl.BlockSpec((1, embed), lambda i: (0, 0)),
            pl.BlockSpec((1, embed), lambda i: (0, 0)),
            pl.BlockSpec((1, embed), lambda i: (0, 0)),
            pl.BlockSpec(W1.shape, lambda i: (0, 0)),
            pl.BlockSpec((1, 2 * embed), lambda i: (0, 0)),
            pl.BlockSpec(W2.shape, lambda i: (0, 0)),
            pl.BlockSpec((1, embed), lambda i: (0, 0)),
            pl.BlockSpec(W3.shape, lambda i: (0, 0)),
            pl.BlockSpec((1, embed // 2), lambda i: (0, 0)),
            pl.BlockSpec(Wp.shape, lambda i: (0, 0)),
            pl.BlockSpec((1, 1), lambda i: (0, 0)),
    ]

    pred = pl.pallas_call(
        functools.partial(_ncf_kernel, n_items=n_items, nb=nb),
        out_shape=jax.ShapeDtypeStruct((batch * n_items, 1), jnp.float32),
        grid=(nb,),
        in_specs=in_specs,
        out_specs=pl.BlockSpec((m_rows, 1), lambda i: (i, 0)),
        scratch_shapes=[
            pltpu.VMEM((2, m_rows, 1, embed), jnp.int32),
            pltpu.VMEM((2, B_BLK, 1, embed), jnp.float32),
            pltpu.VMEM((2, B_BLK, 1, embed), jnp.float32),
            pltpu.VMEM((m_rows, B_BLK), jnp.float32),
            pltpu.SMEM((2, 1, m_rows), jnp.int32),
            pltpu.SMEM((2, 1, B_BLK), jnp.int32),
            pltpu.SemaphoreType.DMA,
            pltpu.SemaphoreType.DMA,
            pltpu.SemaphoreType.DMA((2,)),
            pltpu.SemaphoreType.DMA((2,)),
            pltpu.SemaphoreType.DMA((2,)),
        ],
        compiler_params=_CompilerParams(
            dimension_semantics=("arbitrary",),
        ),
        name="ncf_fused",
    )(item_idx, user_idx, wi_pack, Wu_gmf, Wu_mlp, biases[0], biases[1],
      biases[2], biases[3], W1, biases[4], W2, biases[5], W3, biases[6], Wp, bp2)

    return pred.reshape(batch, n_items)


# speedup vs baseline: 1.1978x; 1.1868x over previous
import functools

import jax
import jax.numpy as jnp
from jax import lax
from jax.experimental import pallas as pl
from jax.experimental.pallas import tpu as pltpu

_CompilerParams = getattr(pltpu, "CompilerParams", None)
if _CompilerParams is None:
    _CompilerParams = pltpu.TPUCompilerParams

_ANY = getattr(pl, "ANY", None)
if _ANY is None:
    _ANY = pltpu.MemorySpace.HBM

B_BLK = 64
_UNROLL = 8


def _ncf_kernel(
    item_idx_ref,
    user_idx_ref,
    wi_ref,
    wug_ref,
    wum_ref,
    bug_ref, bum_ref,
    bi_ref,
    w1_ref, b1_ref, w2_ref, b2_ref, w3_ref, b3_ref, wp_ref, bp_ref,
    out_ref,
    scr_i,
    scr_ug, scr_um,
    idx_smem,
    uidx_smem,
    sem_si, sem_su, sem_i, sem_ug, sem_um,
    *, n_items: int,
):
    m_rows = B_BLK * n_items

    pltpu.make_async_copy(item_idx_ref, idx_smem, sem_si).start()
    pltpu.make_async_copy(user_idx_ref, uidx_smem, sem_su).start()
    pltpu.make_async_copy(item_idx_ref, idx_smem, sem_si).wait()

    def issue_chunk(c, _):
        base = c * _UNROLL
        for i in range(_UNROLL):
            k = base + i
            t = idx_smem[0, 0, k]
            pltpu.make_async_copy(wi_ref.at[t], scr_i.at[k, 0], sem_i).start(
                priority=i % 2)
        return ()
    lax.fori_loop(0, m_rows // _UNROLL, issue_chunk, ())

    pltpu.make_async_copy(user_idx_ref, uidx_smem, sem_su).wait()
    for u in range(B_BLK):
        t = uidx_smem[0, 0, u]
        pltpu.make_async_copy(wug_ref.at[t], scr_ug.at[u, 0], sem_ug).start()
        pltpu.make_async_copy(wum_ref.at[t], scr_um.at[u, 0], sem_um).start()

    pltpu.make_async_copy(scr_ug, scr_ug, sem_ug).wait()
    pltpu.make_async_copy(scr_um, scr_um, sem_um).wait()
    eu_g = scr_ug.reshape(B_BLK, 64)[...] + bug_ref[...]
    eu_m = scr_um.reshape(B_BLK, 64)[...] + bum_ref[...]

    w1 = w1_ref[...]
    u1 = jnp.dot(eu_m, w1[0:64, :], preferred_element_type=jnp.float32)

    k_io = lax.broadcasted_iota(jnp.int32, (m_rows, B_BLK), 0)
    u_io = lax.broadcasted_iota(jnp.int32, (m_rows, B_BLK), 1) * n_items
    r_mat = ((k_io >= u_io) & (k_io < u_io + n_items)).astype(jnp.float32)

    eu_g_rep = jnp.dot(r_mat, eu_g, preferred_element_type=jnp.float32)
    u1_rep = jnp.dot(r_mat, u1, preferred_element_type=jnp.float32)

    w1i_ext = jnp.concatenate([jnp.zeros((64, 128), jnp.float32), w1[64:128, :]], axis=0)

    pltpu.make_async_copy(scr_i, scr_i, sem_i).wait()
    full = scr_i.reshape(m_rows, 128)[...] + bi_ref[...]
    gmf = eu_g_rep * full[:, 0:64]

    i1 = jnp.dot(full, w1i_ext, preferred_element_type=jnp.float32)
    h1 = jnp.maximum(u1_rep + i1 + b1_ref[...], 0.0)
    h2 = jnp.maximum(
        jnp.dot(h1, w2_ref[...], preferred_element_type=jnp.float32) + b2_ref[...], 0.0)
    h3 = jnp.maximum(
        jnp.dot(h2, w3_ref[...], preferred_element_type=jnp.float32) + b3_ref[...], 0.0)

    wp = wp_ref[...]
    logit = (jnp.dot(gmf, wp[0:64, :], preferred_element_type=jnp.float32)
             + jnp.dot(h3, wp[64:96, :], preferred_element_type=jnp.float32)
             + bp_ref[...])
    out_ref[...] = jax.nn.sigmoid(logit)


def kernel(user, item, num_total, Wu_gmf, bu_gmf, Wu_mlp, bu_mlp,
           Wi_gmf, bi_gmf, Wi_mlp, bi_mlp, W1, b1, W2, b2, W3, b3, Wp, bp):
    batch, n_items = item.shape
    nb = batch // B_BLK
    m_rows = B_BLK * n_items
    embed = Wu_gmf.shape[1]

    item_idx = item.astype(jnp.int32).reshape(nb, 1, m_rows)
    user_idx = user.astype(jnp.int32).reshape(nb, 1, B_BLK)

    wi_cat = jnp.concatenate([Wi_gmf, Wi_mlp], axis=1)
    bi_cat = jnp.concatenate([bi_gmf, bi_mlp]).reshape(1, 2 * embed)
    biases = [b.reshape(1, -1) for b in (bu_gmf, bu_mlp, b1, b2, b3)]
    bp2 = bp.reshape(1, 1)

    in_specs = [
            pl.BlockSpec((1, 1, m_rows), lambda i: (i, 0, 0)),
            pl.BlockSpec((1, 1, B_BLK), lambda i: (i, 0, 0)),
            pl.BlockSpec(memory_space=_ANY),
            pl.BlockSpec(memory_space=_ANY),
            pl.BlockSpec(memory_space=_ANY),
            pl.BlockSpec((1, embed), lambda i: (0, 0)),
            pl.BlockSpec((1, embed), lambda i: (0, 0)),
            pl.BlockSpec((1, 2 * embed), lambda i: (0, 0)),
            pl.BlockSpec(W1.shape, lambda i: (0, 0)),
            pl.BlockSpec((1, 2 * embed), lambda i: (0, 0)),
            pl.BlockSpec(W2.shape, lambda i: (0, 0)),
            pl.BlockSpec((1, embed), lambda i: (0, 0)),
            pl.BlockSpec(W3.shape, lambda i: (0, 0)),
            pl.BlockSpec((1, embed // 2), lambda i: (0, 0)),
            pl.BlockSpec(Wp.shape, lambda i: (0, 0)),
            pl.BlockSpec((1, 1), lambda i: (0, 0)),
    ]

    pred = pl.pallas_call(
        functools.partial(_ncf_kernel, n_items=n_items),
        out_shape=jax.ShapeDtypeStruct((batch * n_items, 1), jnp.float32),
        grid=(nb,),
        in_specs=in_specs,
        out_specs=pl.BlockSpec((m_rows, 1), lambda i: (i, 0)),
        scratch_shapes=[
            pltpu.VMEM((m_rows, 1, 2 * embed), jnp.float32),
            pltpu.VMEM((B_BLK, 1, embed), jnp.float32),
            pltpu.VMEM((B_BLK, 1, embed), jnp.float32),
            pltpu.SMEM((1, 1, m_rows), jnp.int32),
            pltpu.SMEM((1, 1, B_BLK), jnp.int32),
            pltpu.SemaphoreType.DMA,
            pltpu.SemaphoreType.DMA,
            pltpu.SemaphoreType.DMA,
            pltpu.SemaphoreType.DMA,
            pltpu.SemaphoreType.DMA,
        ],
        compiler_params=_CompilerParams(
            dimension_semantics=("arbitrary",),
        ),
        name="ncf_fused",
    )(item_idx, user_idx, wi_cat, Wu_gmf, Wu_mlp, biases[0], biases[1],
      bi_cat, W1, biases[2], W2, biases[3], W3, biases[4], Wp, bp2)

    return pred.reshape(batch, n_items)
